# baseline (device time: 361326 ns/iter reference)
import jax
import jax.numpy as jnp
from jax import lax
from jax.experimental import pallas as pl
from jax.experimental.pallas import tpu as pltpu

N_DEV = 8
M_PER = 512
TN = 1024
COMM_DTYPE = jnp.bfloat16


def kernel(x, w_mat, scale_x, scale_w):
    k, n = w_mat.shape
    half = n // 2
    ntiles = half // TN

    x_bf = x.astype(jnp.bfloat16)
    w_bf = w_mat.astype(jnp.bfloat16)
    scale = (scale_x[0] * scale_w[0]).reshape(1, 1).astype(jnp.float32)

    def body(x_ref, w_ref, scale_ref, out_ref,
             buf_r, buf_l,
             send_sems_r, recv_sems_r, send_sems_l, recv_sems_l,
             credit_r, credit_l):
        my = lax.axis_index("i")
        left = lax.rem(my - 1 + N_DEV, N_DEV)
        right = lax.rem(my + 1, N_DEV)

        barrier_sem = pltpu.get_barrier_semaphore()
        for nbr in (left, right):
            pl.semaphore_signal(
                barrier_sem, inc=1,
                device_id=(nbr,), device_id_type=pl.DeviceIdType.MESH,
            )
        pl.semaphore_wait(barrier_sem, 2)

        def gemm_tile(c, off, t):
            xs = x_ref[pl.ds(c * M_PER, M_PER), :]
            return lax.dot_general(
                xs, w_ref[:, pl.ds(off + t * TN, TN)],
                dimension_numbers=(((1,), (0,)), ((), ())),
                preferred_element_type=jnp.float32,
            )

        def make_tile_rdma(bufs, s, src, t, sends, recvs, nbr):
            return pltpu.make_async_remote_copy(
                src_ref=bufs.at[src, t],
                dst_ref=bufs.at[s % 3, t],
                send_sem=sends.at[s % 3, t],
                recv_sem=recvs.at[s % 3, t],
                device_id=(nbr,),
                device_id_type=pl.DeviceIdType.MESH,
            )

        def acc_tile(bufs, s, c, off, t, epilogue):
            p = gemm_tile(c, off, t)
            acc = p + bufs[s % 3, t].astype(jnp.float32)
            if epilogue:
                out_ref[:, pl.ds(off + t * TN, TN)] = jnp.maximum(
                    acc * scale_ref[0, 0], 0.0
                )
            else:
                bufs[s % 3, t] = acc.astype(COMM_DTYPE)

        c0r = lax.rem(my - 1 + N_DEV, N_DEV)
        c0l = lax.rem(my + 1, N_DEV)
        rdmas_r = [[None] * ntiles for _ in range(N_DEV - 1)]
        rdmas_l = [[None] * ntiles for _ in range(N_DEV - 1)]
        for t in range(ntiles):
            buf_r[2, t] = gemm_tile(c0r, 0, t).astype(COMM_DTYPE)
            rdmas_r[0][t] = make_tile_rdma(
                buf_r, 0, 2, t, send_sems_r, recv_sems_r, right)
            rdmas_r[0][t].start()
            buf_l[2, t] = gemm_tile(c0l, half, t).astype(COMM_DTYPE)
            rdmas_l[0][t] = make_tile_rdma(
                buf_l, 0, 2, t, send_sems_l, recv_sems_l, left)
            rdmas_l[0][t].start()

        for s in range(N_DEV - 1):
            if 1 <= s <= 5:
                for t in range(ntiles):
                    rdmas_r[s - 1][t].wait_send()
                    rdmas_l[s - 1][t].wait_send()
                pl.semaphore_signal(
                    credit_r, inc=1,
                    device_id=(left,), device_id_type=pl.DeviceIdType.MESH,
                )
                pl.semaphore_signal(
                    credit_l, inc=1,
                    device_id=(right,), device_id_type=pl.DeviceIdType.MESH,
                )
            cr = lax.rem(my - 2 - s + 2 * N_DEV, N_DEV)
            cl = lax.rem(my + 2 + s, N_DEV)
            if 1 <= s <= 5:
                pl.semaphore_wait(credit_r, 1)
                pl.semaphore_wait(credit_l, 1)
            last = s == N_DEV - 2
            for t in range(ntiles):
                rdmas_r[s][t].wait_recv()
                acc_tile(buf_r, s, cr, 0, t, last)
                if not last:
                    rdmas_r[s + 1][t] = make_tile_rdma(
                        buf_r, s + 1, s % 3, t,
                        send_sems_r, recv_sems_r, right)
                    rdmas_r[s + 1][t].start()
                rdmas_l[s][t].wait_recv()
                acc_tile(buf_l, s, cl, half, t, last)
                if not last:
                    rdmas_l[s + 1][t] = make_tile_rdma(
                        buf_l, s + 1, s % 3, t,
                        send_sems_l, recv_sems_l, left)
                    rdmas_l[s + 1][t].start()
        for d in (rdmas_r, rdmas_l):
            for t in range(ntiles):
                d[N_DEV - 3][t].wait_send()
                d[N_DEV - 2][t].wait_send()

    return pl.pallas_call(
        body,
        out_shape=jax.ShapeDtypeStruct((M_PER, n), jnp.float32),
        in_specs=[
            pl.BlockSpec(memory_space=pltpu.VMEM),
            pl.BlockSpec(memory_space=pltpu.VMEM),
            pl.BlockSpec(memory_space=pltpu.SMEM),
        ],
        out_specs=pl.BlockSpec(memory_space=pltpu.VMEM),
        scratch_shapes=[
            pltpu.VMEM((3, 4, M_PER, TN), COMM_DTYPE),
            pltpu.VMEM((3, 4, M_PER, TN), COMM_DTYPE),
            pltpu.SemaphoreType.DMA((3, 4)),
            pltpu.SemaphoreType.DMA((3, 4)),
            pltpu.SemaphoreType.DMA((3, 4)),
            pltpu.SemaphoreType.DMA((3, 4)),
            pltpu.SemaphoreType.REGULAR,
            pltpu.SemaphoreType.REGULAR,
        ],
        compiler_params=pltpu.CompilerParams(
            collective_id=0, vmem_limit_bytes=100 * 1024 * 1024
        ),
    )(x_bf, w_bf, scale)
